# baseline (speedup 1.0000x reference)
import jax
import jax.numpy as jnp
from jax import lax
from jax.experimental import pallas as pl
from jax.experimental.pallas import tpu as pltpu
from jax.experimental.pallas import tpu_sc as plsc

N_PRIORS = 20000
N_TRUTHS = 256
NP = 20480
NW = 32
PPW = NP // NW
NCH = PPW // 16


def _sc_body(px0_h, py0_h, px1_h, py1_h, pa_h,
             tx0_h, ty0_h, tx1_h, ty1_h, ta_h,
             rm_h, ra_h, bto_h,
             px0_v, py0_v, px1_v, py1_v, pa_v,
             tx0_v, ty0_v, tx1_v, ty1_v, ta_v,
             bto_v, rm_v, ra_v):
    wid = lax.axis_index("s") * 2 + lax.axis_index("c")
    base = wid * PPW
    pltpu.sync_copy(px0_h.at[pl.ds(base, PPW)], px0_v)
    pltpu.sync_copy(py0_h.at[pl.ds(base, PPW)], py0_v)
    pltpu.sync_copy(px1_h.at[pl.ds(base, PPW)], px1_v)
    pltpu.sync_copy(py1_h.at[pl.ds(base, PPW)], py1_v)
    pltpu.sync_copy(pa_h.at[pl.ds(base, PPW)], pa_v)
    pltpu.sync_copy(tx0_h, tx0_v)
    pltpu.sync_copy(ty0_h, ty0_v)
    pltpu.sync_copy(tx1_h, tx1_v)
    pltpu.sync_copy(ty1_h, ty1_v)
    pltpu.sync_copy(ta_h, ta_v)

    lanes = lax.iota(jnp.int32, 16)
    mask0 = lanes == 0
    big = jnp.full((16,), 2 ** 30, jnp.int32)

    def init_chunk(c, _):
        bto_v[pl.ds(c * 16, 16)] = jnp.zeros((16,), jnp.float32)
        return 0

    lax.fori_loop(0, NCH, init_chunk, 0)

    def truth_body(t, _):
        tsplat = jnp.full((16,), t, jnp.int32)
        tx0 = plsc.load_gather(tx0_v, [tsplat])
        ty0 = plsc.load_gather(ty0_v, [tsplat])
        tx1 = plsc.load_gather(tx1_v, [tsplat])
        ty1 = plsc.load_gather(ty1_v, [tsplat])
        ta = plsc.load_gather(ta_v, [tsplat])

        def chunk_body(c, carry):
            best, bchunk = carry
            sl = pl.ds(c * 16, 16)
            ix = jnp.minimum(px1_v[sl], tx1) - jnp.maximum(px0_v[sl], tx0)
            iy = jnp.minimum(py1_v[sl], ty1) - jnp.maximum(py0_v[sl], ty0)
            inter = jnp.maximum(ix, 0.0) * jnp.maximum(iy, 0.0)
            union = (pa_v[sl] + ta) - inter
            ov = inter / union
            bto_v[sl] = jnp.maximum(bto_v[sl], ov)
            better = ov > best
            best = jnp.maximum(best, ov)
            bchunk = jnp.where(better, jnp.full((16,), c, jnp.int32), bchunk)
            return best, bchunk

        best, bchunk = lax.fori_loop(
            0, NCH, chunk_body,
            (jnp.full((16,), -1.0, jnp.float32), jnp.zeros((16,), jnp.int32)))

        m = lax.reduce_max(best, (0,))
        cand = jnp.where(best == m, bchunk * 16 + lanes, big)
        a = lax.reduce_min(cand, (0,)) + base
        plsc.store_scatter(rm_v, [tsplat], jnp.full((16,), m), mask=mask0)
        plsc.store_scatter(ra_v, [tsplat], jnp.full((16,), a), mask=mask0)
        return 0

    lax.fori_loop(0, N_TRUTHS, truth_body, 0)

    pltpu.sync_copy(rm_v, rm_h.at[wid])
    pltpu.sync_copy(ra_v, ra_h.at[wid])
    pltpu.sync_copy(bto_v, bto_h.at[wid])


@jax.jit
def kernel(locs, params, truths):
    pri = jnp.concatenate([locs, params], axis=1).T
    pri = jnp.pad(pri, ((0, 0), (0, NP - N_PRIORS)))
    cx, cy, w, h, _alpha = pri
    px0 = cx - w * 0.5
    py0 = cy - h * 0.5
    px1 = cx + w * 0.5
    py1 = cy + h * 0.5
    pa = w * h
    ta = (truths[:, 2] - truths[:, 0]) * (truths[:, 3] - truths[:, 1])

    mesh = plsc.VectorSubcoreMesh(core_axis_name="c", subcore_axis_name="s",
                                  num_cores=2)
    sc = pl.kernel(
        _sc_body,
        out_type=[
            jax.ShapeDtypeStruct((NW, N_TRUTHS), jnp.float32),
            jax.ShapeDtypeStruct((NW, N_TRUTHS), jnp.int32),
            jax.ShapeDtypeStruct((NW, PPW), jnp.float32),
        ],
        mesh=mesh,
        scratch_types=[
            pltpu.VMEM((PPW,), jnp.float32),
            pltpu.VMEM((PPW,), jnp.float32),
            pltpu.VMEM((PPW,), jnp.float32),
            pltpu.VMEM((PPW,), jnp.float32),
            pltpu.VMEM((PPW,), jnp.float32),
            pltpu.VMEM((N_TRUTHS,), jnp.float32),
            pltpu.VMEM((N_TRUTHS,), jnp.float32),
            pltpu.VMEM((N_TRUTHS,), jnp.float32),
            pltpu.VMEM((N_TRUTHS,), jnp.float32),
            pltpu.VMEM((N_TRUTHS,), jnp.float32),
            pltpu.VMEM((PPW,), jnp.float32),
            pltpu.VMEM((N_TRUTHS,), jnp.float32),
            pltpu.VMEM((N_TRUTHS,), jnp.int32),
        ],
        compiler_params=pltpu.CompilerParams(needs_layout_passes=False),
    )
    rm, ra, bto = sc(px0, py0, px1, py1, pa,
                     truths[:, 0], truths[:, 1], truths[:, 2], truths[:, 3],
                     ta)
    return jnp.sum(rm) + jnp.sum(ra.astype(jnp.float32)) * 0.0 + jnp.sum(bto)


# speedup vs baseline: 8.7553x; 8.7553x over previous
import jax
import jax.numpy as jnp
from jax.experimental import pallas as pl
from jax.experimental.pallas import tpu as pltpu

BETA = 1.0
K = 5.0
THRESH = 0.5
N_PRIORS = 20000
N_TRUTHS = 256

NP = 20480
TB = 8
NBLK = N_TRUTHS // TB
CW = 512
NCHUNK = NP // CW


def _iou_kernel(truths_ref, pri_ref, out_ref,
                px0_ref, py0_ref, px1_ref, py1_ref, pa_ref, bto_ref):
    cx = pri_ref[0:1]
    cy = pri_ref[1:2]
    w = pri_ref[2:3]
    h = pri_ref[3:4]
    px0_ref[...] = jnp.broadcast_to(cx - w * 0.5, (TB, NP))
    py0_ref[...] = jnp.broadcast_to(cy - h * 0.5, (TB, NP))
    px1_ref[...] = jnp.broadcast_to(cx + w * 0.5, (TB, NP))
    py1_ref[...] = jnp.broadcast_to(cy + h * 0.5, (TB, NP))
    pa_ref[...] = jnp.broadcast_to(w * h, (TB, NP))
    bto_ref[...] = jnp.zeros((TB, NP), jnp.float32)

    lane = jax.lax.broadcasted_iota(jnp.int32, (TB, CW), 1)
    sub = jax.lax.broadcasted_iota(jnp.int32, (TB, 1), 0)
    col = jax.lax.broadcasted_iota(jnp.int32, (TB, NBLK), 1)
    big = jnp.int32(2 ** 30)

    def body(r, carry):
        rm_acc, ra_acc = carry
        tb = truths_ref[pl.ds(r * TB, TB), :]
        tx0 = tb[:, 0:1]
        ty0 = tb[:, 1:2]
        tx1 = tb[:, 2:3]
        ty1 = tb[:, 3:4]
        tarea = (tx1 - tx0) * (ty1 - ty0)

        rm = jnp.full((TB, 1), -1.0, jnp.float32)
        ra = jnp.zeros((TB, 1), jnp.int32)
        for c in range(NCHUNK):
            sl = pl.ds(c * CW, CW)
            ix = (jnp.minimum(px1_ref[:, sl], tx1)
                  - jnp.maximum(px0_ref[:, sl], tx0))
            iy = (jnp.minimum(py1_ref[:, sl], ty1)
                  - jnp.maximum(py0_ref[:, sl], ty0))
            inter = jnp.maximum(ix, 0.0) * jnp.maximum(iy, 0.0)
            union = (pa_ref[:, sl] + tarea) - inter
            ov = inter * pl.reciprocal(union, approx=True)

            bto_ref[:, sl] = jnp.maximum(bto_ref[:, sl], ov)
            mc = jnp.max(ov, axis=1, keepdims=True)
            ac = jnp.min(jnp.where(ov == mc, lane, big),
                         axis=1, keepdims=True) + c * CW
            better = mc > rm
            ra = jnp.where(better, ac, ra)
            rm = jnp.maximum(rm, mc)
        hit = col == r
        rm_acc = jnp.where(hit, rm, rm_acc)
        ra_acc = jnp.where(hit, ra, ra_acc)
        return rm_acc, ra_acc

    rm_all, ra_all = jax.lax.fori_loop(
        0, NBLK, body,
        (jnp.zeros((TB, NBLK), jnp.float32), jnp.zeros((TB, NBLK), jnp.int32)),
        unroll=2)

    alpha = pri_ref[4:5]

    num = 0.0
    den = 0.0
    ssum = 0.0
    for c in range(NCHUNK):
        lane_c = lane + c * CW
        key = jnp.full((TB, CW), -1.0, jnp.float32)
        for r in range(NBLK):
            a = ra_all[:, r:r + 1]
            kv = rm_all[:, r:r + 1] + (sub + r * TB).astype(jnp.float32) * 4.0
            key = jnp.where(lane_c == a, kv, key)
        keyn = jnp.max(key, axis=0, keepdims=True)
        matched = keyn >= 0.0
        tsel = jnp.floor(keyn * 0.25)
        lvn = keyn - tsel * 4.0
        bton = jnp.max(bto_ref[:, pl.ds(c * CW, CW)], axis=0, keepdims=True)

        valid = lane_c[0:1] < N_PRIORS
        s = jnp.where(valid, jax.nn.sigmoid(alpha[:, c * CW:(c + 1) * CW]), 0.0)
        xf = jnp.where(matched, K, jnp.where(bton > THRESH, 1.0, 0.0))
        btop = jnp.where(matched, lvn, bton)
        msk = xf > 1e-07
        num += jnp.sum(jnp.where(msk, s * xf * btop, 0.0))
        den += jnp.sum(jnp.where(msk, xf, 0.0))
        ssum += jnp.sum(s)

    out_ref[0, 0] = (num + BETA * ssum) / den


@jax.jit
def kernel(locs, params, truths):
    pri = jnp.concatenate([locs, params], axis=1).T
    pri = jnp.pad(pri, ((0, 0), (0, NP - N_PRIORS)))

    out = pl.pallas_call(
        _iou_kernel,
        in_specs=[
            pl.BlockSpec(memory_space=pltpu.VMEM),
            pl.BlockSpec(memory_space=pltpu.VMEM),
        ],
        out_specs=pl.BlockSpec(memory_space=pltpu.SMEM),
        out_shape=jax.ShapeDtypeStruct((1, 1), jnp.float32),
        scratch_shapes=[
            pltpu.VMEM((TB, NP), jnp.float32),
            pltpu.VMEM((TB, NP), jnp.float32),
            pltpu.VMEM((TB, NP), jnp.float32),
            pltpu.VMEM((TB, NP), jnp.float32),
            pltpu.VMEM((TB, NP), jnp.float32),
            pltpu.VMEM((TB, NP), jnp.float32),
        ],
    )(truths, pri)
    return out[0, 0]
